# baseline (speedup 1.0000x reference)
import functools

import jax
import jax.numpy as jnp
from jax import lax
from jax.experimental import pallas as pl
from jax.experimental.pallas import tpu as pltpu
from jax.experimental.pallas import tpu_sc as plsc

B = 16384
D = 64
NF = 6
NC, NS = 2, 16
NW = NC * NS
R = B // NW
C = 128
NCH = R // C
L = 16
PW = 2 * D

VEC_OFF = (0, 7)
VVEC = 31
PAIR_VB = (100, 31)


def kernel(dayofweek, time, sex, age, month, day,
           W_dayofweek, W_time, W_sex, W_age, W_month, W_day):
    vtbl = jnp.concatenate([W_dayofweek, W_time], axis=0).reshape(-1)

    def pair_table(Wa, Wb):
        va, vb = Wa.shape[0], Wb.shape[0]
        return jnp.concatenate(
            [jnp.repeat(Wa, vb, axis=0), jnp.tile(Wb, (va, 1))], axis=1)

    T1 = pair_table(W_sex, W_age)
    T2 = pair_table(W_month, W_day)

    mesh = plsc.VectorSubcoreMesh(
        core_axis_name="c", subcore_axis_name="s",
        num_cores=NC, num_subcores=NS)

    @functools.partial(
        pl.kernel,
        out_type=jax.ShapeDtypeStruct((B, NF * D), jnp.float32),
        mesh=mesh,
        compiler_params=pltpu.CompilerParams(needs_layout_passes=False),
        scratch_types=[
            pltpu.VMEM((NF * R,), jnp.int32),
            pltpu.VMEM((2 * R,), jnp.int32),
            pltpu.VMEM((VVEC * D,), jnp.float32),
            pltpu.VMEM((2, C, PW), jnp.float32),
            pltpu.VMEM((2, C, PW), jnp.float32),
            pltpu.VMEM((2, C, PW), jnp.float32),
            pltpu.SemaphoreType.DMA,
            pltpu.SemaphoreType.DMA,
            pltpu.SemaphoreType.DMA,
            pltpu.SemaphoreType.DMA,
        ],
    )
    def sck(i0, i1, i2, i3, i4, i5, vt_h, t1, t2,
            out, raw_v, cidx_v, vtbl_v, vasm, sasm1, sasm2,
            g0, g1, w0, w1):
        wid = lax.axis_index("s") * NC + lax.axis_index("c")
        base = wid * R
        idxs = (i0, i1, i2, i3, i4, i5)
        stbls = (t1, t2)
        sasms = (sasm1, sasm2)
        gsem = (g0, g1)
        wsem = (w0, w1)

        bc = pltpu.async_copy(vt_h, vtbl_v, g0)
        stage = [pltpu.async_copy(idxs[f].at[pl.ds(base, R)],
                                  raw_v.at[pl.ds(f * R, R)], g0)
                 for f in range(NF)]
        bc.wait()
        for cp in stage:
            cp.wait()

        for p in range(2):
            vb = PAIR_VB[p]
            for j in range(R // L):
                ia = raw_v[pl.ds((2 + 2 * p) * R + j * L, L)]
                ib = raw_v[pl.ds((3 + 2 * p) * R + j * L, L)]
                cidx_v[pl.ds(p * R + j * L, L)] = ia * vb + ib

        iota = lax.iota(jnp.int32, L)

        def fire(c, s):
            return [pltpu.async_copy(
                stbls[p].at[cidx_v.at[pl.ds(p * R + c * C, C)]],
                sasms[p].at[s], gsem[s]) for p in range(2)]

        def vector_fill(c, s):
            for f in range(2):
                rbase = f * R + c * C

                def body(m, carry, _f=f, _rbase=rbase):
                    j = m >> 2
                    k = m & 3
                    rv = raw_v[pl.ds(_rbase + j * L, L)] + VEC_OFF[_f]
                    ldbase = rv * D + k * L
                    rowv = j * L + iota
                    colp = iota
                    v = plsc.load_gather(vtbl_v, [ldbase + colp])
                    for d in range(L):
                        if d + 1 < L:
                            ncolp = (iota + (d + 1)) & (L - 1)
                            nv = plsc.load_gather(vtbl_v,
                                                  [ldbase + ncolp])
                        plsc.store_scatter(
                            vasm.at[s],
                            [rowv, _f * D + k * L + colp], v)
                        if d + 1 < L:
                            colp, v = ncolp, nv
                    return carry

                lax.fori_loop(0, (C // L) * (D // L), body, 0)

        writes = [None] * NCH
        gath = fire(0, 0)
        for c in range(NCH):
            s = c % 2
            if c + 1 < NCH:
                if c >= 1:
                    for cp in writes[c - 1]:
                        cp.wait()
                ngath = fire(c + 1, 1 - s)
            vector_fill(c, s)
            for cp in gath:
                cp.wait()
            rows = pl.ds(base + c * C, C)
            writes[c] = [
                pltpu.async_copy(vasm.at[s],
                                 out.at[rows, pl.ds(0, PW)], wsem[s]),
                pltpu.async_copy(sasm1.at[s],
                                 out.at[rows, pl.ds(PW, PW)], wsem[s]),
                pltpu.async_copy(sasm2.at[s],
                                 out.at[rows, pl.ds(2 * PW, PW)], wsem[s]),
            ]
            if c + 1 < NCH:
                gath = ngath
        for cp in writes[NCH - 2]:
            cp.wait()
        for cp in writes[NCH - 1]:
            cp.wait()

    return sck(dayofweek.astype(jnp.int32), time.astype(jnp.int32),
               sex.astype(jnp.int32), age.astype(jnp.int32),
               month.astype(jnp.int32), day.astype(jnp.int32),
               vtbl, T1, T2)


# speedup vs baseline: 1.0346x; 1.0346x over previous
import functools

import jax
import jax.numpy as jnp
from jax import lax
from jax.experimental import pallas as pl
from jax.experimental.pallas import tpu as pltpu
from jax.experimental.pallas import tpu_sc as plsc

B = 16384
D = 64
NF = 6
NC, NS = 2, 16
NW = NC * NS
R = B // NW
C = 128
NCH = R // C
L = 16
PW = 2 * D

VEC_FEATS = ((0, 0, 0, 0), (1, 7, 0, 1), (4, 31, 1, 0), (5, 43, 1, 1))
VVEC = 74
SEX_AGE_VB = 100


def kernel(dayofweek, time, sex, age, month, day,
           W_dayofweek, W_time, W_sex, W_age, W_month, W_day):
    vtbl = jnp.concatenate([W_dayofweek, W_time, W_month, W_day],
                           axis=0).reshape(-1)
    T1 = jnp.concatenate(
        [jnp.repeat(W_sex, W_age.shape[0], axis=0),
         jnp.tile(W_age, (W_sex.shape[0], 1))], axis=1)

    mesh = plsc.VectorSubcoreMesh(
        core_axis_name="c", subcore_axis_name="s",
        num_cores=NC, num_subcores=NS)

    @functools.partial(
        pl.kernel,
        out_type=jax.ShapeDtypeStruct((B, NF * D), jnp.float32),
        mesh=mesh,
        compiler_params=pltpu.CompilerParams(needs_layout_passes=False),
        scratch_types=[
            pltpu.VMEM((NF * R,), jnp.int32),
            pltpu.VMEM((R,), jnp.int32),
            pltpu.VMEM((VVEC * D,), jnp.float32),
            pltpu.VMEM((2, C, PW), jnp.float32),
            pltpu.VMEM((2, C, PW), jnp.float32),
            pltpu.VMEM((2, C, PW), jnp.float32),
            pltpu.SemaphoreType.DMA,
            pltpu.SemaphoreType.DMA,
            pltpu.SemaphoreType.DMA,
            pltpu.SemaphoreType.DMA,
        ],
    )
    def sck(i0, i1, i2, i3, i4, i5, vt_h, t1,
            out, raw_v, cidx_v, vtbl_v, vasm0, sasm, vasm1,
            g0, g1, w0, w1):
        wid = lax.axis_index("s") * NC + lax.axis_index("c")
        base = wid * R
        idxs = (i0, i1, i2, i3, i4, i5)
        vasms = (vasm0, vasm1)
        gsem = (g0, g1)
        wsem = (w0, w1)

        bc = pltpu.async_copy(vt_h, vtbl_v, g0)
        stage = [pltpu.async_copy(idxs[f].at[pl.ds(base, R)],
                                  raw_v.at[pl.ds(f * R, R)], g0)
                 for f in range(NF)]
        bc.wait()
        for cp in stage:
            cp.wait()

        for j in range(R // L):
            ia = raw_v[pl.ds(2 * R + j * L, L)]
            ib = raw_v[pl.ds(3 * R + j * L, L)]
            cidx_v[pl.ds(j * L, L)] = ia * SEX_AGE_VB + ib

        iota = lax.iota(jnp.int32, L)

        def fire(c, s):
            return pltpu.async_copy(
                t1.at[cidx_v.at[pl.ds(c * C, C)]], sasm.at[s], gsem[s])

        def vector_fill(c, s):
            for slot, off, blk, half in VEC_FEATS:
                rbase = slot * R + c * C
                dst = vasms[blk]

                def body(m, carry, _off=off, _rbase=rbase,
                         _dst=dst, _half=half):
                    j = m >> 2
                    k = m & 3
                    rv = raw_v[pl.ds(_rbase + j * L, L)] + _off
                    ldbase = rv * D + k * L
                    rowv = j * L + iota
                    colp = iota
                    v = plsc.load_gather(vtbl_v, [ldbase + colp])
                    for d in range(L):
                        if d + 1 < L:
                            ncolp = (iota + (d + 1)) & (L - 1)
                            nv = plsc.load_gather(vtbl_v,
                                                  [ldbase + ncolp])
                        plsc.store_scatter(
                            _dst.at[s],
                            [rowv, _half * D + k * L + colp], v)
                        if d + 1 < L:
                            colp, v = ncolp, nv
                    return carry

                lax.fori_loop(0, (C // L) * (D // L), body, 0)

        writes = [None] * NCH
        gath = fire(0, 0)
        for c in range(NCH):
            s = c % 2
            if c + 1 < NCH:
                if c >= 1:
                    for cp in writes[c - 1]:
                        cp.wait()
                ngath = fire(c + 1, 1 - s)
            vector_fill(c, s)
            gath.wait()
            rows = pl.ds(base + c * C, C)
            writes[c] = [
                pltpu.async_copy(vasm0.at[s],
                                 out.at[rows, pl.ds(0, PW)], wsem[s]),
                pltpu.async_copy(sasm.at[s],
                                 out.at[rows, pl.ds(PW, PW)], wsem[s]),
                pltpu.async_copy(vasm1.at[s],
                                 out.at[rows, pl.ds(2 * PW, PW)], wsem[s]),
            ]
            if c + 1 < NCH:
                gath = ngath
        for cp in writes[NCH - 2]:
            cp.wait()
        for cp in writes[NCH - 1]:
            cp.wait()

    return sck(dayofweek.astype(jnp.int32), time.astype(jnp.int32),
               sex.astype(jnp.int32), age.astype(jnp.int32),
               month.astype(jnp.int32), day.astype(jnp.int32),
               vtbl, T1)


# speedup vs baseline: 1.0420x; 1.0071x over previous
import functools

import jax
import jax.numpy as jnp
from jax import lax
from jax.experimental import pallas as pl
from jax.experimental.pallas import tpu as pltpu
from jax.experimental.pallas import tpu_sc as plsc

B = 16384
D = 64
NF = 6
NC, NS = 2, 16
NW = NC * NS
R = B // NW
C = 128
NCH = R // C
L = 16
PW = 2 * D

VEC_FEATS = ((0, 0, 0, 0), (1, 7, 0, 1), (4, 31, 1, 0), (5, 43, 1, 1))
VVEC = 74
SEX_AGE_VB = 100


def kernel(dayofweek, time, sex, age, month, day,
           W_dayofweek, W_time, W_sex, W_age, W_month, W_day):
    vtbl = jnp.concatenate([W_dayofweek, W_time, W_month, W_day],
                           axis=0).reshape(-1)
    T1 = jnp.concatenate(
        [jnp.repeat(W_sex, W_age.shape[0], axis=0),
         jnp.tile(W_age, (W_sex.shape[0], 1))], axis=1)

    mesh = plsc.VectorSubcoreMesh(
        core_axis_name="c", subcore_axis_name="s",
        num_cores=NC, num_subcores=NS)

    @functools.partial(
        pl.kernel,
        out_type=jax.ShapeDtypeStruct((B, NF * D), jnp.float32),
        mesh=mesh,
        compiler_params=pltpu.CompilerParams(needs_layout_passes=False),
        scratch_types=[
            pltpu.VMEM((NF * R,), jnp.int32),
            pltpu.VMEM((R,), jnp.int32),
            pltpu.VMEM((VVEC * D,), jnp.float32),
            pltpu.VMEM((2, C, PW), jnp.float32),
            pltpu.VMEM((2, C, PW), jnp.float32),
            pltpu.VMEM((2, C, PW), jnp.float32),
            pltpu.SemaphoreType.DMA,
            pltpu.SemaphoreType.DMA,
            pltpu.SemaphoreType.DMA,
            pltpu.SemaphoreType.DMA,
        ],
    )
    def sck(i0, i1, i2, i3, i4, i5, vt_h, t1,
            out, raw_v, cidx_v, vtbl_v, vasm0, sasm, vasm1,
            g0, g1, w0, w1):
        wid = lax.axis_index("s") * NC + lax.axis_index("c")
        base = wid * R
        idxs = (i0, i1, i2, i3, i4, i5)
        vasms = (vasm0, vasm1)
        gsem = (g0, g1)
        wsem = (w0, w1)

        stage_sa = [pltpu.async_copy(idxs[f].at[pl.ds(base, R)],
                                     raw_v.at[pl.ds(f * R, R)], g1)
                    for f in (2, 3)]
        bc = pltpu.async_copy(vt_h, vtbl_v, g0)
        stage = [pltpu.async_copy(idxs[f].at[pl.ds(base, R)],
                                  raw_v.at[pl.ds(f * R, R)], g0)
                 for f in (0, 1, 4, 5)]
        for cp in stage_sa:
            cp.wait()

        for j in range(R // L):
            ia = raw_v[pl.ds(2 * R + j * L, L)]
            ib = raw_v[pl.ds(3 * R + j * L, L)]
            cidx_v[pl.ds(j * L, L)] = ia * SEX_AGE_VB + ib

        bc.wait()
        for cp in stage:
            cp.wait()

        iota = lax.iota(jnp.int32, L)

        def fire(c, s):
            return pltpu.async_copy(
                t1.at[cidx_v.at[pl.ds(c * C, C)]], sasm.at[s], gsem[s])

        def vector_fill(c, s):
            for slot, off, blk, half in VEC_FEATS:
                rbase = slot * R + c * C
                dst = vasms[blk]

                def body(m, carry, _off=off, _rbase=rbase,
                         _dst=dst, _half=half):
                    j = m >> 2
                    k = m & 3
                    rv = raw_v[pl.ds(_rbase + j * L, L)] + _off
                    ldbase = rv * D + k * L
                    rowv = j * L + iota
                    colp = iota
                    v = plsc.load_gather(vtbl_v, [ldbase + colp])
                    for d in range(L):
                        if d + 1 < L:
                            ncolp = (iota + (d + 1)) & (L - 1)
                            nv = plsc.load_gather(vtbl_v,
                                                  [ldbase + ncolp])
                        plsc.store_scatter(
                            _dst.at[s],
                            [rowv, _half * D + k * L + colp], v)
                        if d + 1 < L:
                            colp, v = ncolp, nv
                    return carry

                lax.fori_loop(0, (C // L) * (D // L), body, 0)

        writes = [None] * NCH
        gath = fire(0, 0)
        for c in range(NCH):
            s = c % 2
            if c + 1 < NCH:
                if c >= 1:
                    for cp in writes[c - 1]:
                        cp.wait()
                ngath = fire(c + 1, 1 - s)
            vector_fill(c, s)
            rows = pl.ds(base + c * C, C)
            writes[c] = [
                pltpu.async_copy(vasm0.at[s],
                                 out.at[rows, pl.ds(0, PW)], wsem[s]),
                pltpu.async_copy(vasm1.at[s],
                                 out.at[rows, pl.ds(2 * PW, PW)], wsem[s]),
            ]
            gath.wait()
            writes[c].append(
                pltpu.async_copy(sasm.at[s],
                                 out.at[rows, pl.ds(PW, PW)], wsem[s]))
            if c + 1 < NCH:
                gath = ngath
        for cp in writes[NCH - 2]:
            cp.wait()
        for cp in writes[NCH - 1]:
            cp.wait()

    return sck(dayofweek.astype(jnp.int32), time.astype(jnp.int32),
               sex.astype(jnp.int32), age.astype(jnp.int32),
               month.astype(jnp.int32), day.astype(jnp.int32),
               vtbl, T1)
